# Initial kernel scaffold; baseline (speedup 1.0000x reference)
import jax
import jax.numpy as jnp
from jax.experimental import pallas as pl


def kernel(embeds, edge_index, trend, edge_weight, users, pos_items, neg_items):
    raise NotImplementedError("write your pallas kernel here")


# speedup vs baseline: 2.8553x; 2.8553x over previous
import functools

import jax
import jax.numpy as jnp
from jax import lax
from jax.experimental import pallas as pl
from jax.experimental.pallas import tpu as pltpu
from jax.experimental.pallas import tpu_sc as plsc

N_USERS = 25000
N_NODES = 50000
D = 64
E = 800000
BATCH = 4096

CHUNK = 128
NCHUNKS = E // CHUNK
NS = 16
NC = 2
HALF = N_NODES // NC
ACC_ROWS = 25088
STRIPE = ACC_ROWS // NS
CHUNKS_PER_TILE = -(-NCHUNKS // NS)

_mesh = plsc.VectorSubcoreMesh(core_axis_name="c", subcore_axis_name="s")

_BCAST_DN = lax.GatherDimensionNumbers(
    offset_dims=(), collapsed_slice_dims=(0,), start_index_map=(0,))


def _lane_bcast(v16, e):
    idx = jnp.full((16, 1), e, dtype=jnp.int32)
    return lax.gather(v16, idx, _BCAST_DN, slice_sizes=(1,),
                      mode=lax.GatherScatterMode.PROMISE_IN_BOUNDS)


@functools.partial(
    pl.kernel,
    mesh=_mesh,
    out_type=jax.ShapeDtypeStruct((N_NODES, D), jnp.float32),
    compiler_params=pltpu.CompilerParams(use_tc_tiling_on_sc=False),
    scratch_types=[
        pltpu.VMEM((CHUNK,), jnp.int32),
        pltpu.VMEM((CHUNK,), jnp.int32),
        pltpu.VMEM((CHUNK,), jnp.float32),
        pltpu.VMEM((CHUNK,), jnp.int32),
        pltpu.VMEM((CHUNK, D), jnp.float32),
        pltpu.VMEM_SHARED((ACC_ROWS, D), jnp.float32),
        pltpu.SemaphoreType.DMA,
    ],
)
def _hop(agg_in, rowh, colh, trendh, zerosh, out,
         rowv, colv, trendv, lcv, rv, acc, sem):
    c = lax.axis_index("c")
    s = lax.axis_index("s")
    core_off = c * HALF

    pltpu.sync_copy(zerosh, acc.at[pl.ds(s * STRIPE, STRIPE)])
    plsc.subcore_barrier()

    def chunk_body(j, carry):
        ci = s + j * NS

        @pl.when(ci < NCHUNKS)
        def _():
            b = ci * CHUNK
            pltpu.sync_copy(rowh.at[pl.ds(b, CHUNK)], rowv)
            pltpu.sync_copy(colh.at[pl.ds(b, CHUNK)], colv)
            pltpu.sync_copy(trendh.at[pl.ds(b, CHUNK)], trendv)
            cp = pltpu.async_copy(agg_in.at[rowv], rv, sem)
            for g in range(CHUNK // 16):
                c16 = colv[pl.ds(g * 16, 16)]
                lc = c16 - core_off
                ok = (lc >= 0) & (lc < HALF)
                lcv[pl.ds(g * 16, 16)] = jnp.where(ok, lc, HALF)
            cp.wait()
            for g in range(CHUNK // 16):
                t16 = trendv[pl.ds(g * 16, 16)]
                for e in range(16):
                    row = g * 16 + e
                    bc = _lane_bcast(t16, e)
                    for q in range(D // 16):
                        sl = pl.ds(q * 16, 16)
                        rv[row, sl] = rv[row, sl] * bc
            pltpu.sync_copy(rv, acc.at[lcv], add=True)

        return carry

    lax.fori_loop(0, CHUNKS_PER_TILE, chunk_body, 0)
    plsc.subcore_barrier()

    ob = jnp.minimum(s * STRIPE, HALF - STRIPE)
    pltpu.sync_copy(acc.at[pl.ds(ob, STRIPE)],
                    out.at[pl.ds(core_off + ob, STRIPE)])


_PB = BATCH // (NC * NS)


@functools.partial(
    pl.kernel,
    mesh=_mesh,
    out_type=tuple(jax.ShapeDtypeStruct((BATCH, D), jnp.float32)
                   for _ in range(6)),
    compiler_params=pltpu.CompilerParams(use_tc_tiling_on_sc=False),
    scratch_types=[
        pltpu.VMEM((_PB,), jnp.int32),
        pltpu.VMEM((_PB,), jnp.int32),
        pltpu.VMEM((_PB, D), jnp.float32),
        pltpu.VMEM((_PB, D), jnp.float32),
        pltpu.VMEM((_PB, D), jnp.float32),
        pltpu.VMEM((_PB, D), jnp.float32),
        pltpu.VMEM((_PB, D), jnp.float32),
        pltpu.SemaphoreType.DMA,
    ],
)
def _final(embeds, e1, e2, e3, users, pos, neg,
           o_u, o_p, o_n, o_ue, o_pe, o_ne,
           idxv, pidxv, A, B, C, Dd, P, sem):
    c = lax.axis_index("c")
    s = lax.axis_index("s")
    b0 = (s * NC + c) * _PB

    def pool_into_P():
        def body(i, carry):
            for q in range(D // 16):
                sl = pl.ds(q * 16, 16)
                a = A[i, sl]
                b = B[i, sl]
                cc = C[i, sl]
                d = Dd[i, sl]
                P[i, sl] = (a + b + cc + d) * 0.25
            return carry
        lax.fori_loop(0, _PB, body, 0)

    def gather4(iref):
        cps = [pltpu.async_copy(t.at[iref], dst, sem)
               for t, dst in ((embeds, A), (e1, B), (e2, C), (e3, Dd))]
        for cp in cps:
            cp.wait()

    pltpu.sync_copy(users.at[pl.ds(b0, _PB)], idxv)
    gather4(idxv)
    pool_into_P()
    pltpu.sync_copy(P, o_u.at[pl.ds(b0, _PB)])
    pltpu.sync_copy(A, o_ue.at[pl.ds(b0, _PB)])

    for src, o_pool, o_raw in ((pos, o_p, o_pe), (neg, o_n, o_ne)):
        pltpu.sync_copy(src.at[pl.ds(b0, _PB)], idxv)
        cp = pltpu.async_copy(embeds.at[idxv], A, sem)
        for g in range(_PB // 16):
            i16 = idxv[pl.ds(g * 16, 16)]
            pidxv[pl.ds(g * 16, 16)] = HALF + lax.rem(i16, HALF)
        cp.wait()
        pltpu.sync_copy(A, o_raw.at[pl.ds(b0, _PB)])
        gather4(pidxv)
        pool_into_P()
        pltpu.sync_copy(P, o_pool.at[pl.ds(b0, _PB)])


def kernel(embeds, edge_index, trend, edge_weight, users, pos_items, neg_items):
    row = edge_index[0]
    col = edge_index[1]
    zeros = jnp.zeros((STRIPE, D), jnp.float32)
    e1 = _hop(embeds, row, col, trend, zeros)
    e2 = _hop(e1, row, col, trend, zeros)
    e3 = _hop(e2, row, col, trend, zeros)
    u, p, n, ue, pe, ne = _final(embeds, e1, e2, e3,
                                 users, pos_items, neg_items)
    return (u, p, n, ue, pe, ne)
